# initial kernel scaffold (unmeasured)
import jax
import jax.numpy as jnp
from jax import lax
from jax.experimental import pallas as pl
from jax.experimental.pallas import tpu as pltpu

N_DEV = 4
M_BLK = 2048
M_HALF = 1024
D = 2048
F_SHARD = 8192
F_CHUNK = 512
N_FC = F_SHARD // F_CHUNK

BF16 = jnp.bfloat16
F32 = jnp.float32


def kernel(x, W1, W2):
    xb = x.astype(BF16)
    w1b = W1.astype(BF16)
    w2b = W2.astype(BF16)

    def body(x_hbm, w1_hbm, w2_hbm, out_ref,
             ag_hbm, rs_hbm,
             xstage, w1buf, w2buf, acc, rs_send,
             ag_send_sems, ag_recv_sems, rs_send_sems, rs_recv_sems,
             w1sem, w2sem, local_sem):
        me = lax.axis_index("i")

        barrier = pltpu.get_barrier_semaphore()
        for d in range(1, N_DEV):
            pl.semaphore_signal(
                barrier, inc=1,
                device_id=((me + d) % N_DEV,),
                device_id_type=pl.DeviceIdType.MESH,
            )
        pl.semaphore_wait(barrier, N_DEV - 1)

        ag_sends = []
        for d in range(1, N_DEV):
            rdma = pltpu.make_async_remote_copy(
                src_ref=x_hbm,
                dst_ref=ag_hbm.at[d - 1],
                send_sem=ag_send_sems.at[d - 1],
                recv_sem=ag_recv_sems.at[d - 1],
                device_id=((me + d) % N_DEV,),
                device_id_type=pl.DeviceIdType.MESH,
            )
            rdma.start()
            ag_sends.append(rdma)

        def compute_block_into_acc():
            acc[...] = jnp.zeros_like(acc)

            def fc_body(fc, carry):
                c1 = pltpu.make_async_copy(
                    w1_hbm.at[:, pl.ds(fc * F_CHUNK, F_CHUNK)], w1buf, w1sem)
                c2 = pltpu.make_async_copy(
                    w2_hbm.at[pl.ds(fc * F_CHUNK, F_CHUNK), :], w2buf, w2sem)
                c1.start()
                c2.start()
                c1.wait()
                c2.wait()
                for half in range(2):
                    xh = xstage[pl.ds(half * M_HALF, M_HALF), :]
                    h = jnp.dot(xh, w1buf[...], preferred_element_type=F32)
                    h = h * jax.nn.sigmoid(h)
                    p = jnp.dot(h.astype(BF16), w2buf[...],
                                preferred_element_type=F32)
                    acc[half] = acc[half] + p
                return carry

            lax.fori_loop(0, N_FC, fc_body, 0)

        rs_inflight = [None, None]
        for d in (0, 1, 3, 2):
            if d == 0:
                cp = pltpu.make_async_copy(x_hbm, xstage, local_sem)
                cp.start()
                cp.wait()
            else:
                slot = 3 - d
                recv = pltpu.make_async_remote_copy(
                    src_ref=x_hbm,
                    dst_ref=ag_hbm.at[slot],
                    send_sem=ag_send_sems.at[slot],
                    recv_sem=ag_recv_sems.at[slot],
                    device_id=(me,),
                    device_id_type=pl.DeviceIdType.MESH,
                )
                recv.wait_recv()
                cp = pltpu.make_async_copy(ag_hbm.at[slot], xstage, local_sem)
                cp.start()
                cp.wait()

            compute_block_into_acc()

            if d == 0:
                for half in range(2):
                    out_ref[pl.ds(half * M_HALF, M_HALF), :] = (
                        acc[half].astype(BF16))
            else:
                for half in range(2):
                    if rs_inflight[half] is not None:
                        rs_inflight[half].wait_send()
                    rs_send[half] = acc[half].astype(BF16)
                    rdma = pltpu.make_async_remote_copy(
                        src_ref=rs_send.at[half],
                        dst_ref=rs_hbm.at[d - 1,
                                          pl.ds(half * M_HALF, M_HALF), :],
                        send_sem=rs_send_sems.at[half],
                        recv_sem=rs_recv_sems.at[d - 1, half],
                        device_id=((me + d) % N_DEV,),
                        device_id_type=pl.DeviceIdType.MESH,
                    )
                    rdma.start()
                    rs_inflight[half] = rdma

        for half in range(2):
            if rs_inflight[half] is not None:
                rs_inflight[half].wait_send()

        for half in range(2):
            accv = out_ref[pl.ds(half * M_HALF, M_HALF), :].astype(F32)
            for s in range(3):
                recv = pltpu.make_async_remote_copy(
                    src_ref=rs_send.at[half],
                    dst_ref=rs_hbm.at[s, pl.ds(half * M_HALF, M_HALF), :],
                    send_sem=rs_send_sems.at[half],
                    recv_sem=rs_recv_sems.at[s, half],
                    device_id=(me,),
                    device_id_type=pl.DeviceIdType.MESH,
                )
                recv.wait_recv()
                cp = pltpu.make_async_copy(
                    rs_hbm.at[s, pl.ds(half * M_HALF, M_HALF), :],
                    rs_send.at[half], local_sem)
                cp.start()
                cp.wait()
                accv = accv + rs_send[half].astype(F32)
            out_ref[pl.ds(half * M_HALF, M_HALF), :] = accv.astype(BF16)

        for rdma in ag_sends:
            rdma.wait_send()

    return pl.pallas_call(
        body,
        out_shape=jax.ShapeDtypeStruct((M_BLK, D), BF16),
        in_specs=[
            pl.BlockSpec(memory_space=pltpu.HBM),
            pl.BlockSpec(memory_space=pltpu.HBM),
            pl.BlockSpec(memory_space=pltpu.HBM),
        ],
        out_specs=pl.BlockSpec(memory_space=pltpu.VMEM),
        scratch_shapes=[
            pltpu.HBM((N_DEV - 1, M_BLK, D), BF16),
            pltpu.HBM((N_DEV - 1, M_BLK, D), BF16),
            pltpu.VMEM((M_BLK, D), BF16),
            pltpu.VMEM((D, F_CHUNK), BF16),
            pltpu.VMEM((F_CHUNK, D), BF16),
            pltpu.VMEM((2, M_HALF, D), F32),
            pltpu.VMEM((2, M_HALF, D), BF16),
            pltpu.SemaphoreType.DMA((N_DEV - 1,)),
            pltpu.SemaphoreType.DMA((N_DEV - 1,)),
            pltpu.SemaphoreType.DMA((2,)),
            pltpu.SemaphoreType.DMA((N_DEV - 1, 2)),
            pltpu.SemaphoreType.DMA,
            pltpu.SemaphoreType.DMA,
            pltpu.SemaphoreType.DMA,
        ],
        compiler_params=pltpu.CompilerParams(collective_id=0),
    )(xb, w1b, w2b)


# baseline (device time: 969440 ns/iter reference)
import jax
import jax.numpy as jnp
from jax import lax
from jax.experimental import pallas as pl
from jax.experimental.pallas import tpu as pltpu

N_DEV = 4
M_BLK = 2048
M_HALF = 1024
D = 2048
F_SHARD = 8192
F_CHUNK = 512
N_FC = F_SHARD // F_CHUNK

BF16 = jnp.bfloat16
F32 = jnp.float32


def kernel(x, W1, W2):
    xb = x.astype(BF16)
    w1b = W1.astype(BF16)
    w2b = W2.astype(BF16)

    def body(x_hbm, w1_hbm, w2_hbm, out_ref, ag_hbm, rs_hbm,
             xstage, w1buf, w2buf, acc, rs_send,
             ag_send_sems, ag_recv_sems, rs_send_sems, rs_recv_sems,
             w1sem, w2sem, local_sem):
        me = lax.axis_index("i")

        barrier = pltpu.get_barrier_semaphore()
        for d in range(1, N_DEV):
            pl.semaphore_signal(
                barrier, inc=1,
                device_id=((me + d) % N_DEV,),
                device_id_type=pl.DeviceIdType.MESH,
            )
        pl.semaphore_wait(barrier, N_DEV - 1)

        ag_sends = []
        for d in range(1, N_DEV):
            rdma = pltpu.make_async_remote_copy(
                src_ref=x_hbm,
                dst_ref=ag_hbm.at[d - 1],
                send_sem=ag_send_sems.at[d - 1],
                recv_sem=ag_recv_sems.at[d - 1],
                device_id=((me + d) % N_DEV,),
                device_id_type=pl.DeviceIdType.MESH,
            )
            rdma.start()
            ag_sends.append(rdma)

        def compute_block_into_acc():
            acc[...] = jnp.zeros_like(acc)

            def fc_body(fc, carry):
                c1 = pltpu.make_async_copy(
                    w1_hbm.at[:, pl.ds(fc * F_CHUNK, F_CHUNK)], w1buf, w1sem)
                c2 = pltpu.make_async_copy(
                    w2_hbm.at[pl.ds(fc * F_CHUNK, F_CHUNK), :], w2buf, w2sem)
                c1.start()
                c2.start()
                c1.wait()
                c2.wait()
                for half in range(2):
                    xh = xstage[pl.ds(half * M_HALF, M_HALF), :]
                    h = jnp.dot(xh, w1buf[...], preferred_element_type=F32)
                    h = h * jax.nn.sigmoid(h)
                    p = jnp.dot(h.astype(BF16), w2buf[...],
                                preferred_element_type=F32)
                    acc[half] = acc[half] + p
                return carry

            lax.fori_loop(0, N_FC, fc_body, 0)

        rs_inflight = [None, None]
        for d in (0, 1, 3, 2):
            if d == 0:
                cp = pltpu.make_async_copy(x_hbm, xstage, local_sem)
                cp.start()
                cp.wait()
            else:
                slot = 3 - d
                recv = pltpu.make_async_remote_copy(
                    src_ref=x_hbm,
                    dst_ref=ag_hbm.at[slot],
                    send_sem=ag_send_sems.at[slot],
                    recv_sem=ag_recv_sems.at[slot],
                    device_id=(me,),
                    device_id_type=pl.DeviceIdType.MESH,
                )
                recv.wait_recv()
                cp = pltpu.make_async_copy(ag_hbm.at[slot], xstage, local_sem)
                cp.start()
                cp.wait()

            compute_block_into_acc()

            if d == 0:
                for half in range(2):
                    out_ref[pl.ds(half * M_HALF, M_HALF), :] = (
                        acc[half].astype(BF16))
            else:
                for half in range(2):
                    if rs_inflight[half] is not None:
                        rs_inflight[half].wait_send()
                    rs_send[half] = acc[half].astype(BF16)
                    rdma = pltpu.make_async_remote_copy(
                        src_ref=rs_send.at[half],
                        dst_ref=rs_hbm.at[d - 1,
                                          pl.ds(half * M_HALF, M_HALF), :],
                        send_sem=rs_send_sems.at[half],
                        recv_sem=rs_recv_sems.at[d - 1, half],
                        device_id=((me + d) % N_DEV,),
                        device_id_type=pl.DeviceIdType.MESH,
                    )
                    rdma.start()
                    rs_inflight[half] = rdma

        for half in range(2):
            if rs_inflight[half] is not None:
                rs_inflight[half].wait_send()

        for half in range(2):
            accv = out_ref[pl.ds(half * M_HALF, M_HALF), :].astype(F32)
            for s in range(3):
                recv = pltpu.make_async_remote_copy(
                    src_ref=rs_send.at[half],
                    dst_ref=rs_hbm.at[s, pl.ds(half * M_HALF, M_HALF), :],
                    send_sem=rs_send_sems.at[half],
                    recv_sem=rs_recv_sems.at[s, half],
                    device_id=(me,),
                    device_id_type=pl.DeviceIdType.MESH,
                )
                recv.wait_recv()
                cp = pltpu.make_async_copy(
                    rs_hbm.at[s, pl.ds(half * M_HALF, M_HALF), :],
                    rs_send.at[half], local_sem)
                cp.start()
                cp.wait()
                accv = accv + rs_send[half].astype(F32)
            out_ref[pl.ds(half * M_HALF, M_HALF), :] = accv.astype(BF16)

        for rdma in ag_sends:
            rdma.wait_send()

    out, _, _ = pl.pallas_call(
        body,
        out_shape=(
            jax.ShapeDtypeStruct((M_BLK, D), BF16),
            jax.ShapeDtypeStruct((N_DEV - 1, M_BLK, D), BF16),
            jax.ShapeDtypeStruct((N_DEV - 1, M_BLK, D), BF16),
        ),
        in_specs=[
            pl.BlockSpec(memory_space=pltpu.HBM),
            pl.BlockSpec(memory_space=pltpu.HBM),
            pl.BlockSpec(memory_space=pltpu.HBM),
        ],
        out_specs=(
            pl.BlockSpec(memory_space=pltpu.VMEM),
            pl.BlockSpec(memory_space=pltpu.HBM),
            pl.BlockSpec(memory_space=pltpu.HBM),
        ),
        scratch_shapes=[
            pltpu.VMEM((M_BLK, D), BF16),
            pltpu.VMEM((D, F_CHUNK), BF16),
            pltpu.VMEM((F_CHUNK, D), BF16),
            pltpu.VMEM((2, M_HALF, D), F32),
            pltpu.VMEM((2, M_HALF, D), BF16),
            pltpu.SemaphoreType.DMA((N_DEV - 1,)),
            pltpu.SemaphoreType.DMA((N_DEV - 1,)),
            pltpu.SemaphoreType.DMA((2,)),
            pltpu.SemaphoreType.DMA((N_DEV - 1, 2)),
            pltpu.SemaphoreType.DMA,
            pltpu.SemaphoreType.DMA,
            pltpu.SemaphoreType.DMA,
        ],
        compiler_params=pltpu.CompilerParams(
            collective_id=0,
            vmem_limit_bytes=60 * 1024 * 1024,
        ),
    )(xb, w1b, w2b)
    return out


# device time: 818085 ns/iter; 1.1850x vs baseline; 1.1850x over previous
import jax
import jax.numpy as jnp
from jax import lax
from jax.experimental import pallas as pl
from jax.experimental.pallas import tpu as pltpu

N_DEV = 4
M_BLK = 2048
M_HALF = 1024
D = 2048
F_SHARD = 8192
F_CHUNK = 512
N_FC = F_SHARD // F_CHUNK

BF16 = jnp.bfloat16
F32 = jnp.float32


def kernel(x, W1, W2):
    xb = x.astype(BF16)
    w1b = W1.astype(BF16)
    w2b = W2.astype(BF16)

    def body(x_hbm, w1_hbm, w2_hbm, out_ref, ag_hbm, rs_hbm,
             xstage, w1buf, w2buf, acc, rs_send,
             ag_send_sems, ag_recv_sems, rs_send_sems, rs_recv_sems,
             w1sems, w2sems, local_sem):
        me = lax.axis_index("i")

        barrier = pltpu.get_barrier_semaphore()
        for d in range(1, N_DEV):
            pl.semaphore_signal(
                barrier, inc=1,
                device_id=((me + d) % N_DEV,),
                device_id_type=pl.DeviceIdType.MESH,
            )
        pl.semaphore_wait(barrier, N_DEV - 1)

        ag_sends = []
        for d in range(1, N_DEV):
            rdma = pltpu.make_async_remote_copy(
                src_ref=x_hbm,
                dst_ref=ag_hbm.at[d - 1],
                send_sem=ag_send_sems.at[d - 1],
                recv_sem=ag_recv_sems.at[d - 1],
                device_id=((me + d) % N_DEV,),
                device_id_type=pl.DeviceIdType.MESH,
            )
            rdma.start()
            ag_sends.append(rdma)

        def start_w(fc, slot):
            pltpu.make_async_copy(
                w1_hbm.at[:, pl.ds(fc * F_CHUNK, F_CHUNK)],
                w1buf.at[slot], w1sems.at[slot]).start()
            pltpu.make_async_copy(
                w2_hbm.at[pl.ds(fc * F_CHUNK, F_CHUNK), :],
                w2buf.at[slot], w2sems.at[slot]).start()

        def wait_w(slot):
            pltpu.make_async_copy(
                w1_hbm.at[:, pl.ds(0, F_CHUNK)],
                w1buf.at[slot], w1sems.at[slot]).wait()
            pltpu.make_async_copy(
                w2_hbm.at[pl.ds(0, F_CHUNK), :],
                w2buf.at[slot], w2sems.at[slot]).wait()

        def compute_half_into_acc(half):
            acc[...] = jnp.zeros_like(acc)
            start_w(0, 0)

            def fc_body(fc, carry):
                cur = lax.rem(fc, 2)
                nxt = 1 - cur

                @pl.when(fc + 1 < N_FC)
                def _():
                    start_w(fc + 1, nxt)

                wait_w(cur)
                xh = xstage[pl.ds(half * M_HALF, M_HALF), :]
                h = jnp.dot(xh, w1buf[cur], preferred_element_type=F32)
                h = h.astype(BF16)
                h = h * jax.nn.sigmoid(h)
                p = jnp.dot(h, w2buf[cur], preferred_element_type=F32)
                acc[...] = acc[...] + p
                return carry

            lax.fori_loop(0, N_FC, fc_body, 0)

        rs_inflight = [None, None]
        for d in (0, 1, 2, 3):
            if d == 0:
                cp = pltpu.make_async_copy(x_hbm, xstage, local_sem)
                cp.start()
                cp.wait()
            else:
                slot = 3 - d
                recv = pltpu.make_async_remote_copy(
                    src_ref=x_hbm,
                    dst_ref=ag_hbm.at[slot],
                    send_sem=ag_send_sems.at[slot],
                    recv_sem=ag_recv_sems.at[slot],
                    device_id=(me,),
                    device_id_type=pl.DeviceIdType.MESH,
                )
                recv.wait_recv()
                cp = pltpu.make_async_copy(ag_hbm.at[slot], xstage, local_sem)
                cp.start()
                cp.wait()

            for half in range(2):
                compute_half_into_acc(half)
                if d == 0:
                    out_ref[pl.ds(half * M_HALF, M_HALF), :] = (
                        acc[...].astype(BF16))
                else:
                    if rs_inflight[half] is not None:
                        rs_inflight[half].wait_send()
                    rs_send[half] = acc[...].astype(BF16)
                    rdma = pltpu.make_async_remote_copy(
                        src_ref=rs_send.at[half],
                        dst_ref=rs_hbm.at[d - 1,
                                          pl.ds(half * M_HALF, M_HALF), :],
                        send_sem=rs_send_sems.at[half],
                        recv_sem=rs_recv_sems.at[d - 1, half],
                        device_id=((me + d) % N_DEV,),
                        device_id_type=pl.DeviceIdType.MESH,
                    )
                    rdma.start()
                    rs_inflight[half] = rdma

        for half in range(2):
            if rs_inflight[half] is not None:
                rs_inflight[half].wait_send()

        for half in range(2):
            accv = out_ref[pl.ds(half * M_HALF, M_HALF), :].astype(F32)
            for s in range(3):
                recv = pltpu.make_async_remote_copy(
                    src_ref=rs_send.at[half],
                    dst_ref=rs_hbm.at[s, pl.ds(half * M_HALF, M_HALF), :],
                    send_sem=rs_send_sems.at[half],
                    recv_sem=rs_recv_sems.at[s, half],
                    device_id=(me,),
                    device_id_type=pl.DeviceIdType.MESH,
                )
                recv.wait_recv()
                cp = pltpu.make_async_copy(
                    rs_hbm.at[s, pl.ds(half * M_HALF, M_HALF), :],
                    rs_send.at[half], local_sem)
                cp.start()
                cp.wait()
                accv = accv + rs_send[half].astype(F32)
            out_ref[pl.ds(half * M_HALF, M_HALF), :] = accv.astype(BF16)

        for rdma in ag_sends:
            rdma.wait_send()

    out, _, _ = pl.pallas_call(
        body,
        out_shape=(
            jax.ShapeDtypeStruct((M_BLK, D), BF16),
            jax.ShapeDtypeStruct((N_DEV - 1, M_BLK, D), BF16),
            jax.ShapeDtypeStruct((N_DEV - 1, M_BLK, D), BF16),
        ),
        in_specs=[
            pl.BlockSpec(memory_space=pltpu.HBM),
            pl.BlockSpec(memory_space=pltpu.HBM),
            pl.BlockSpec(memory_space=pltpu.HBM),
        ],
        out_specs=(
            pl.BlockSpec(memory_space=pltpu.VMEM),
            pl.BlockSpec(memory_space=pltpu.HBM),
            pl.BlockSpec(memory_space=pltpu.HBM),
        ),
        scratch_shapes=[
            pltpu.VMEM((M_BLK, D), BF16),
            pltpu.VMEM((2, D, F_CHUNK), BF16),
            pltpu.VMEM((2, F_CHUNK, D), BF16),
            pltpu.VMEM((M_HALF, D), F32),
            pltpu.VMEM((2, M_HALF, D), BF16),
            pltpu.SemaphoreType.DMA((N_DEV - 1,)),
            pltpu.SemaphoreType.DMA((N_DEV - 1,)),
            pltpu.SemaphoreType.DMA((2,)),
            pltpu.SemaphoreType.DMA((N_DEV - 1, 2)),
            pltpu.SemaphoreType.DMA((2,)),
            pltpu.SemaphoreType.DMA((2,)),
            pltpu.SemaphoreType.DMA,
        ],
        compiler_params=pltpu.CompilerParams(
            collective_id=0,
            vmem_limit_bytes=60 * 1024 * 1024,
        ),
    )(xb, w1b, w2b)
    return out
